# baseline (speedup 1.0000x reference)
import functools
import jax
import jax.numpy as jnp
from jax import lax
from jax.experimental import pallas as pl
from jax.experimental.pallas import tpu as pltpu
from jax.experimental.pallas import tpu_sc as plsc

_D = 1024
_H = 4096
_E = 8
_K = 2
_N = 2048
_B = 256
_HB = 512
_NHB = _H // _HB
_P = _N * _K
_G = _P // _B + _E
_ROWS = _G * _B

_INTERP = False


def _gate_kernel(x_ref, gw_ref, gb_ref, w_ref, i_ref):
    logits = jnp.dot(x_ref[...], gw_ref[...],
                     preferred_element_type=jnp.float32) + gb_ref[...]
    cols = jax.lax.broadcasted_iota(jnp.int32, logits.shape, 1)
    m1 = jnp.max(logits, axis=1)
    i1 = jnp.argmax(logits, axis=1).astype(jnp.int32)
    masked = jnp.where(cols == i1[:, None], -jnp.inf, logits)
    m2 = jnp.max(masked, axis=1)
    i2 = jnp.argmax(masked, axis=1).astype(jnp.int32)
    e2 = jnp.exp(m2 - m1)
    w1 = 1.0 / (1.0 + e2)
    w2 = e2 / (1.0 + e2)
    w_ref[...] = jnp.stack([w1, w2], axis=1)
    i_ref[...] = jnp.stack([i1, i2], axis=1)


def _gate(x, gate_w, gate_b):
    bt = 256
    return pl.pallas_call(
        _gate_kernel,
        grid=(_N // bt,),
        in_specs=[
            pl.BlockSpec((bt, _D), lambda t: (t, 0)),
            pl.BlockSpec((_D, _E), lambda t: (0, 0)),
            pl.BlockSpec((_E,), lambda t: (0,)),
        ],
        out_specs=[
            pl.BlockSpec((bt, _K), lambda t: (t, 0)),
            pl.BlockSpec((bt, _K), lambda t: (t, 0)),
        ],
        out_shape=[
            jax.ShapeDtypeStruct((_N, _K), jnp.float32),
            jax.ShapeDtypeStruct((_N, _K), jnp.int32),
        ],
        interpret=_INTERP,
    )(x, gate_w, gate_b)


def _route(idx, w):
    idxf = idx.reshape(-1)
    onehot = (idxf[:, None] == jnp.arange(_E)[None, :]).astype(jnp.int32)
    counts = onehot.sum(0)
    pc = ((counts + _B - 1) // _B) * _B
    ends = jnp.cumsum(pc)
    off = ends - pc
    ranks = jnp.cumsum(onehot, 0) - onehot
    r = (ranks * onehot).sum(1)
    pos = off[idxf] + r
    src = jnp.zeros((_ROWS,), jnp.int32).at[pos].set(
        jnp.arange(_P, dtype=jnp.int32) // _K)
    ws = jnp.zeros((_ROWS,), jnp.float32).at[pos].set(w.reshape(-1))
    te = jnp.minimum(
        jnp.searchsorted(ends, jnp.arange(_G, dtype=jnp.int32) * _B,
                         side='right').astype(jnp.int32),
        _E - 1)
    return te, src, ws, pos.reshape(_N, _K)


def _gelu(a):
    return a * 0.5 * (1.0 + jax.lax.erf(a * 0.7071067811865476))


def _ffn_kernel(te_ref, xs_ref, w1_ref, b1_ref, w2_ref, b2_ref, ws_ref,
                out_ref):
    h = pl.program_id(1)
    a = jnp.dot(xs_ref[...], w1_ref[0],
                preferred_element_type=jnp.float32) + b1_ref[0]
    y = jnp.dot(_gelu(a), w2_ref[0], preferred_element_type=jnp.float32)

    @pl.when(h == 0)
    def _():
        out_ref[...] = jnp.zeros_like(out_ref)

    out_ref[...] += y

    @pl.when(h == _NHB - 1)
    def _():
        out_ref[...] = (out_ref[...] + b2_ref[0]) * ws_ref[...]


def _ffn(te, xs, W1, b1, W2, b2, ws):
    grid_spec = pltpu.PrefetchScalarGridSpec(
        num_scalar_prefetch=1,
        grid=(_G, _NHB),
        in_specs=[
            pl.BlockSpec((_B, _D), lambda g, h, te: (g, 0)),
            pl.BlockSpec((1, _D, _HB), lambda g, h, te: (te[g], 0, h)),
            pl.BlockSpec((1, 1, _HB), lambda g, h, te: (te[g], 0, h)),
            pl.BlockSpec((1, _HB, _D), lambda g, h, te: (te[g], h, 0)),
            pl.BlockSpec((1, 1, _D), lambda g, h, te: (te[g], 0, 0)),
            pl.BlockSpec((_B, 1), lambda g, h, te: (g, 0)),
        ],
        out_specs=pl.BlockSpec((_B, _D), lambda g, h, te: (g, 0)),
    )
    return pl.pallas_call(
        _ffn_kernel,
        grid_spec=grid_spec,
        out_shape=jax.ShapeDtypeStruct((_ROWS, _D), jnp.float32),
        interpret=_INTERP,
    )(te, xs, W1, b1.reshape(_E, 1, _H), W2, b2.reshape(_E, 1, _D),
      ws.reshape(_ROWS, 1))


_NW = 32
_RPW = _ROWS // _NW
_GCH = 64
_TPW = _N // _NW
_TCH = 32


def _sc_mesh():
    return plsc.VectorSubcoreMesh(core_axis_name="c", subcore_axis_name="s")


def _gather_sc(x, src):
    @functools.partial(
        pl.kernel, mesh=_sc_mesh(),
        out_type=jax.ShapeDtypeStruct((_ROWS, _D), jnp.float32),
        scratch_types=[
            pltpu.VMEM((_RPW,), jnp.int32),
            pltpu.VMEM((_GCH, _D), jnp.float32),
            pltpu.SemaphoreType.DMA,
        ],
    )
    def k(x_hbm, src_hbm, xs_hbm, idx_v, rows_v, sem):
        wid = lax.axis_index("s") * 2 + lax.axis_index("c")
        base = wid * _RPW
        pltpu.sync_copy(src_hbm.at[pl.ds(base, _RPW)], idx_v)
        for c in range(_RPW // _GCH):
            pltpu.async_copy(
                x_hbm.at[idx_v.at[pl.ds(c * _GCH, _GCH)]], rows_v, sem
            ).wait()
            pltpu.sync_copy(rows_v, xs_hbm.at[pl.ds(base + c * _GCH, _GCH)])

    return k(x, src)


def _combine_sc(ys, p0, p1):
    @functools.partial(
        pl.kernel, mesh=_sc_mesh(),
        out_type=jax.ShapeDtypeStruct((_N, _D), jnp.float32),
        scratch_types=[
            pltpu.VMEM((_TPW,), jnp.int32),
            pltpu.VMEM((_TPW,), jnp.int32),
            pltpu.VMEM((_TCH, _D), jnp.float32),
            pltpu.VMEM((_TCH, _D), jnp.float32),
            pltpu.SemaphoreType.DMA,
            pltpu.SemaphoreType.DMA,
        ],
    )
    def k(ys_hbm, p0_hbm, p1_hbm, out_hbm, i0_v, i1_v, a_v, b_v, s0, s1):
        wid = lax.axis_index("s") * 2 + lax.axis_index("c")
        base = wid * _TPW
        pltpu.sync_copy(p0_hbm.at[pl.ds(base, _TPW)], i0_v)
        pltpu.sync_copy(p1_hbm.at[pl.ds(base, _TPW)], i1_v)
        for c in range(_TPW // _TCH):
            cp0 = pltpu.async_copy(
                ys_hbm.at[i0_v.at[pl.ds(c * _TCH, _TCH)]], a_v, s0)
            cp1 = pltpu.async_copy(
                ys_hbm.at[i1_v.at[pl.ds(c * _TCH, _TCH)]], b_v, s1)
            cp0.wait()
            cp1.wait()

            def body(r, carry):
                for j in range(_D // 16):
                    sl = pl.ds(j * 16, 16)
                    a_v[r, sl] = a_v[r, sl] + b_v[r, sl]
                return carry

            lax.fori_loop(0, _TCH, body, 0)
            pltpu.sync_copy(a_v, out_hbm.at[pl.ds(base + c * _TCH, _TCH)])

    return k(ys, p0, p1)


def kernel(x, gate_w, gate_b, W1, b1, W2, b2):
    w, idx = _gate(x, gate_w, gate_b)
    te, src, ws, pos = _route(idx, w)
    xs = _gather_sc(x, src)
    ys = _ffn(te, xs, W1, b1, W2, b2, ws)
    out = _combine_sc(ys, pos[:, 0], pos[:, 1])
    return out


# speedup vs baseline: 1.2876x; 1.2876x over previous
import functools
import jax
import jax.numpy as jnp
from jax import lax
from jax.experimental import pallas as pl
from jax.experimental.pallas import tpu as pltpu
from jax.experimental.pallas import tpu_sc as plsc

_D = 1024
_H = 4096
_E = 8
_K = 2
_N = 2048
_B = 256
_HS = 2
_P = _N * _K
_G = _P // _B + _E
_ROWS = _G * _B

_INTERP = False


def _gate_kernel(x_ref, gw_ref, gb_ref, w_ref, i_ref):
    logits = jnp.dot(x_ref[...], gw_ref[...],
                     preferred_element_type=jnp.float32) + gb_ref[...]
    cols = jax.lax.broadcasted_iota(jnp.int32, logits.shape, 1)
    m1 = jnp.max(logits, axis=1)
    i1 = jnp.argmax(logits, axis=1).astype(jnp.int32)
    masked = jnp.where(cols == i1[:, None], -jnp.inf, logits)
    m2 = jnp.max(masked, axis=1)
    i2 = jnp.argmax(masked, axis=1).astype(jnp.int32)
    e2 = jnp.exp(m2 - m1)
    w1 = 1.0 / (1.0 + e2)
    w2 = e2 / (1.0 + e2)
    w_ref[...] = jnp.stack([w1, w2], axis=1)
    i_ref[...] = jnp.stack([i1, i2], axis=1)


def _gate(x, gate_w, gate_b):
    bt = 256
    return pl.pallas_call(
        _gate_kernel,
        grid=(_N // bt,),
        in_specs=[
            pl.BlockSpec((bt, _D), lambda t: (t, 0)),
            pl.BlockSpec((_D, _E), lambda t: (0, 0)),
            pl.BlockSpec((_E,), lambda t: (0,)),
        ],
        out_specs=[
            pl.BlockSpec((bt, _K), lambda t: (t, 0)),
            pl.BlockSpec((bt, _K), lambda t: (t, 0)),
        ],
        out_shape=[
            jax.ShapeDtypeStruct((_N, _K), jnp.float32),
            jax.ShapeDtypeStruct((_N, _K), jnp.int32),
        ],
        interpret=_INTERP,
    )(x, gate_w, gate_b)


def _route(idx, w):
    idxf = idx.reshape(-1)
    onehot = (idxf[:, None] == jnp.arange(_E)[None, :]).astype(jnp.int32)
    counts = onehot.sum(0)
    pc = ((counts + _B - 1) // _B) * _B
    ends = jnp.cumsum(pc)
    off = ends - pc
    ranks = jnp.cumsum(onehot, 0) - onehot
    r = (ranks * onehot).sum(1)
    pos = off[idxf] + r
    src = jnp.zeros((_ROWS,), jnp.int32).at[pos].set(
        jnp.arange(_P, dtype=jnp.int32) // _K)
    ws = jnp.zeros((_ROWS,), jnp.float32).at[pos].set(w.reshape(-1))
    n_used = jnp.sum(pc) // _B
    te_raw = jnp.searchsorted(ends, jnp.arange(_G, dtype=jnp.int32) * _B,
                              side='right').astype(jnp.int32)
    te = jnp.minimum(te_raw, te_raw[n_used - 1])
    return te, n_used.reshape(1).astype(jnp.int32), src, ws, pos.reshape(_N, _K)


def _gelu(a):
    return a * 0.5 * (1.0 + jax.lax.erf(a * 0.7071067811865476))


def _ffn_kernel(te_ref, nu_ref, xs_ref, w1_ref, b1_ref, w2_ref, b2_ref,
                ws_ref, out_ref):
    hs = pl.program_id(0)
    g = pl.program_id(1)

    @pl.when(g < nu_ref[0])
    def _():
        xb = xs_ref[...].astype(jnp.bfloat16)
        a = jnp.dot(xb, w1_ref[0].astype(jnp.bfloat16),
                    preferred_element_type=jnp.float32) + b1_ref[0]
        y = jnp.dot(_gelu(a).astype(jnp.bfloat16),
                    w2_ref[0].astype(jnp.bfloat16),
                    preferred_element_type=jnp.float32)
        scale = jnp.where(hs == 0, 1.0, 0.0)
        out_ref[...] = (y + scale * b2_ref[0]) * ws_ref[...]


def _ffn(te, n_used, xs, W1, b1, W2, b2, ws):
    hh = _H // _HS
    grid_spec = pltpu.PrefetchScalarGridSpec(
        num_scalar_prefetch=2,
        grid=(_HS, _G),
        in_specs=[
            pl.BlockSpec((_B, _D), lambda hs, g, te, nu: (g, 0)),
            pl.BlockSpec((1, _D, hh), lambda hs, g, te, nu: (te[g], 0, hs)),
            pl.BlockSpec((1, 1, hh), lambda hs, g, te, nu: (te[g], 0, hs)),
            pl.BlockSpec((1, hh, _D), lambda hs, g, te, nu: (te[g], hs, 0)),
            pl.BlockSpec((1, 1, _D), lambda hs, g, te, nu: (te[g], 0, 0)),
            pl.BlockSpec((_B, 1), lambda hs, g, te, nu: (g, 0)),
        ],
        out_specs=pl.BlockSpec((_B, _D), lambda hs, g, te, nu: (hs * _G + g, 0)),
    )
    return pl.pallas_call(
        _ffn_kernel,
        grid_spec=grid_spec,
        out_shape=jax.ShapeDtypeStruct((_HS * _ROWS, _D), jnp.float32),
        interpret=_INTERP,
    )(te, n_used, xs, W1, b1.reshape(_E, 1, _H), W2,
      b2.reshape(_E, 1, _D), ws.reshape(_ROWS, 1))


_NW = 32
_RPW = _ROWS // _NW
_GCH = 64
_TPW = _N // _NW
_TCH = 16


def _sc_mesh():
    return plsc.VectorSubcoreMesh(core_axis_name="c", subcore_axis_name="s")


def _gather_sc(x, src):
    @functools.partial(
        pl.kernel, mesh=_sc_mesh(),
        out_type=jax.ShapeDtypeStruct((_ROWS, _D), jnp.float32),
        scratch_types=[
            pltpu.VMEM((_RPW,), jnp.int32),
            pltpu.VMEM((_GCH, _D), jnp.float32),
            pltpu.SemaphoreType.DMA,
        ],
    )
    def k(x_hbm, src_hbm, xs_hbm, idx_v, rows_v, sem):
        wid = lax.axis_index("s") * 2 + lax.axis_index("c")
        base = wid * _RPW
        pltpu.sync_copy(src_hbm.at[pl.ds(base, _RPW)], idx_v)
        for c in range(_RPW // _GCH):
            pltpu.async_copy(
                x_hbm.at[idx_v.at[pl.ds(c * _GCH, _GCH)]], rows_v, sem
            ).wait()
            pltpu.sync_copy(rows_v, xs_hbm.at[pl.ds(base + c * _GCH, _GCH)])

    return k(x, src)


def _combine_sc(ys, p0, p1):
    @functools.partial(
        pl.kernel, mesh=_sc_mesh(),
        out_type=jax.ShapeDtypeStruct((_N, _D), jnp.float32),
        scratch_types=[
            pltpu.VMEM((_TPW,), jnp.int32),
            pltpu.VMEM((_TPW,), jnp.int32),
            pltpu.VMEM((_TCH, _D), jnp.float32),
            pltpu.VMEM((_TCH, _D), jnp.float32),
            pltpu.VMEM((_TCH, _D), jnp.float32),
            pltpu.VMEM((_TCH, _D), jnp.float32),
            pltpu.SemaphoreType.DMA,
            pltpu.SemaphoreType.DMA,
            pltpu.SemaphoreType.DMA,
            pltpu.SemaphoreType.DMA,
        ],
    )
    def k(ys_hbm, p0_hbm, p1_hbm, out_hbm, i0_v, i1_v, a_v, b_v, c_v, d_v,
          s0, s1, s2, s3):
        wid = lax.axis_index("s") * 2 + lax.axis_index("c")
        base = wid * _TPW
        pltpu.sync_copy(p0_hbm.at[pl.ds(base, _TPW)], i0_v)
        pltpu.sync_copy(p1_hbm.at[pl.ds(base, _TPW)], i1_v)
        for c in range(_TPW // _TCH):
            isl = pl.ds(c * _TCH, _TCH)
            cps = [
                pltpu.async_copy(ys_hbm.at[i0_v.at[isl]], a_v, s0),
                pltpu.async_copy(ys_hbm.at[i1_v.at[isl]], b_v, s1),
                pltpu.async_copy(ys_hbm.at[i0_v[isl] + _ROWS], c_v, s2),
                pltpu.async_copy(ys_hbm.at[i1_v[isl] + _ROWS], d_v, s3),
            ]
            for cp in cps:
                cp.wait()

            def body(r, carry):
                for j in range(_D // 16):
                    sl = pl.ds(j * 16, 16)
                    a_v[r, sl] = ((a_v[r, sl] + b_v[r, sl])
                                  + (c_v[r, sl] + d_v[r, sl]))
                return carry

            lax.fori_loop(0, _TCH, body, 0)
            pltpu.sync_copy(a_v, out_hbm.at[pl.ds(base + c * _TCH, _TCH)])

    return k(ys, p0, p1)


def kernel(x, gate_w, gate_b, W1, b1, W2, b2):
    w, idx = _gate(x, gate_w, gate_b)
    te, n_used, src, ws, pos = _route(idx, w)
    xs = _gather_sc(x, src)
    ys = _ffn(te, n_used, xs, W1, b1, W2, b2, ws)
    out = _combine_sc(ys, pos[:, 0], pos[:, 1])
    return out
